# baseline (speedup 1.0000x reference)
import functools
import math

import jax
import jax.numpy as jnp
from jax.experimental import pallas as pl
from jax.experimental.pallas import tpu as pltpu
from jax.experimental.pallas import tpu_sc as plsc

B_, N_, K_, C_ = 8, 4096, 16, 128
ROWS = B_ * N_
NIDX = ROWS * K_
NEG_INV_S = -1.0 / math.sqrt(C_)

RB = 2048
PB = 512
NBLK = ROWS // PB
OB = 4096
GW = 128


def _proj_body(f_ref, idx_ref, wk_ref, wv_ref, bv_ref, kv_ref, idxo_ref):
    f = f_ref[...]
    kf = jnp.dot(f, wk_ref[...].T, preferred_element_type=jnp.float32)
    vf = jnp.dot(f, wv_ref[...].T, preferred_element_type=jnp.float32)
    kb = jax.lax.bitcast_convert_type(kf * NEG_INV_S, jnp.int32)
    vb = jax.lax.bitcast_convert_type(vf + bv_ref[...], jnp.int32)
    hi = (kb + jnp.int32(0x8000)) & jnp.int32(-65536)
    lo = jax.lax.shift_right_logical(vb + jnp.int32(0x8000), 16)
    kv_ref[...] = hi | lo
    i = pl.program_id(0)
    base = (i * RB) // N_ * N_
    idxo_ref[...] = idx_ref[...].T + base


def _attn_body(g_ref, wf_ref, bf_ref, h_ref, ps_ref, pq_ref):
    w = g_ref[0]
    lg = jax.lax.bitcast_convert_type(w & jnp.int32(-65536), jnp.float32)
    v = jax.lax.bitcast_convert_type(
        jax.lax.shift_left(w, jnp.int32(16)), jnp.float32)
    e = jnp.exp(lg)
    s = jnp.sum(e, axis=0)
    t = jnp.max((e - s[None]) * v, axis=0)
    ctx = t / s
    h = jnp.dot(ctx, wf_ref[...].T, preferred_element_type=jnp.float32)
    h = h + bf_ref[...]
    h_ref[...] = h.astype(jnp.bfloat16)
    ps_ref[...] = jnp.sum(h, axis=0, keepdims=True)[None]
    pq_ref[...] = jnp.sum(h * h, axis=0, keepdims=True)[None]


def _attn_body_alias(g_ref, wf_ref, bf_ref, hp_ref, pp_ref, qp_ref,
                     h_ref, ps_ref, pq_ref):
    del hp_ref, pp_ref, qp_ref
    _attn_body(g_ref, wf_ref, bf_ref, h_ref, ps_ref, pq_ref)


def _stats_body(ps_ref, pq_ref, gm_ref, bt_ref, sc_ref, sh_ref):
    tot = jnp.sum(ps_ref[...], axis=0)
    tot2 = jnp.sum(pq_ref[...], axis=0)
    mean = tot / ROWS
    var = tot2 / ROWS - mean * mean
    sc = gm_ref[...] / jnp.sqrt(var + 1e-5)
    sc_ref[...] = sc
    sh_ref[...] = bt_ref[...] - mean * sc


def _out_body(h_ref, f_ref, sc_ref, sh_ref, o_ref):
    hn = h_ref[...].astype(jnp.float32) * sc_ref[...] + sh_ref[...]
    o_ref[...] = f_ref[...] + jnp.where(hn >= 0, hn, 0.2 * hn)


NC = 4
CROWS = ROWS // NC


NW = 32
WIDX = K_ * CROWS // NW
GB = 256
NSTEP = WIDX // GB


def _sc_gather(kv, idxt):
    mesh = plsc.VectorSubcoreMesh(core_axis_name="core",
                                  subcore_axis_name="subcore")

    npb = CROWS // PB
    nbuf = 3

    @functools.partial(
        pl.kernel,
        out_type=jax.ShapeDtypeStruct((npb, K_, PB, C_), jnp.int32),
        mesh=mesh,
        scratch_types=[
            pltpu.VMEM((WIDX,), jnp.int32),
            pltpu.VMEM((nbuf, GB, C_), jnp.int32),
            pltpu.SemaphoreType.DMA,
            pltpu.SemaphoreType.DMA,
            pltpu.SemaphoreType.DMA,
        ])
    def gk(kv_hbm, i_hbm, o_hbm, idx_v, bufs, isem, gsem, osem):
        wid = jax.lax.axis_index("subcore") * 2 + jax.lax.axis_index("core")
        wpk = CROWS // WIDX
        k = wid // wpk
        j0 = (wid % wpk) * WIDX
        pltpu.async_copy(
            i_hbm.at[k, pl.ds(j0, WIDX)], idx_v, isem).wait()

        def mk_g(t):
            return pltpu.make_async_copy(
                kv_hbm.at[idx_v.at[pl.ds(t * GB, GB)]],
                bufs.at[t % nbuf], gsem)

        def mk_o(t):
            pos = j0 + t * GB
            return pltpu.make_async_copy(
                bufs.at[t % nbuf],
                o_hbm.at[pos // PB, k, pl.ds(pos % PB, GB)], osem)

        gs = [None] * NSTEP
        os_ = [None] * NSTEP
        for t in range(NSTEP):
            if t >= nbuf:
                os_[t - nbuf].wait()
            gs[t] = mk_g(t)
            gs[t].start()
            if t >= 1:
                gs[t - 1].wait()
                os_[t - 1] = mk_o(t - 1)
                os_[t - 1].start()
        gs[NSTEP - 1].wait()
        os_[NSTEP - 1] = mk_o(NSTEP - 1)
        os_[NSTEP - 1].start()
        for t in range(max(0, NSTEP - nbuf + 1), NSTEP):
            os_[t].wait()

    return gk(kv, idxt)


def kernel(features, idx, pos, Wq, bq, Wk, bk, Wv, bv, Wf, bf, gamma, beta):
    f2 = features.reshape(ROWS, C_)
    idxt = idx.reshape(ROWS, K_).astype(jnp.int32)

    cblk = CROWS // RB
    proj_chunk = [
        pl.pallas_call(
            _proj_body,
            grid=(cblk,),
            in_specs=[
                pl.BlockSpec((RB, C_), lambda i, c=c: (i + c * cblk, 0)),
                pl.BlockSpec((RB, K_), lambda i, c=c: (i + c * cblk, 0)),
                pl.BlockSpec((C_, C_), lambda i: (0, 0)),
                pl.BlockSpec((C_, C_), lambda i: (0, 0)),
                pl.BlockSpec((1, C_), lambda i: (0, 0)),
            ],
            out_specs=[
                pl.BlockSpec((RB, C_), lambda i: (i, 0)),
                pl.BlockSpec((K_, RB), lambda i: (0, i)),
            ],
            out_shape=[
                jax.ShapeDtypeStruct((CROWS, C_), jnp.int32),
                jax.ShapeDtypeStruct((K_, CROWS), jnp.int32),
            ],
            compiler_params=pltpu.CompilerParams(
                dimension_semantics=("parallel",)),
        )
        for c in range(NC)
    ]

    NBC = NBLK // NC
    attn_out_shape = [
        jax.ShapeDtypeStruct((ROWS, C_), jnp.bfloat16),
        jax.ShapeDtypeStruct((NBLK, 1, C_), jnp.float32),
        jax.ShapeDtypeStruct((NBLK, 1, C_), jnp.float32),
    ]

    def attn_specs(c):
        return dict(
            grid=(NBC,),
            out_specs=[
                pl.BlockSpec((PB, C_), lambda i, c=c: (i + c * NBC, 0)),
                pl.BlockSpec((1, 1, C_), lambda i, c=c: (i + c * NBC, 0, 0)),
                pl.BlockSpec((1, 1, C_), lambda i, c=c: (i + c * NBC, 0, 0)),
            ],
            out_shape=attn_out_shape,
            compiler_params=pltpu.CompilerParams(
                dimension_semantics=("parallel",)),
        )

    base_in = [
        pl.BlockSpec((1, K_, PB, C_), lambda i: (i, 0, 0, 0)),
        pl.BlockSpec((C_, C_), lambda i: (0, 0)),
        pl.BlockSpec((1, C_), lambda i: (0, 0)),
    ]
    prev_in = [
        pl.BlockSpec((8, C_), lambda i: (0, 0)),
        pl.BlockSpec((1, 1, C_), lambda i: (0, 0, 0)),
        pl.BlockSpec((1, 1, C_), lambda i: (0, 0, 0)),
    ]
    h = ps = pq = None
    bfr = bf.reshape(1, C_)
    for c in range(NC):
        kv_c, idxo_c = proj_chunk[c](f2, idxt, Wk, Wv, bv.reshape(1, C_))
        g_c = _sc_gather(kv_c, idxo_c)
        if c == 0:
            h, ps, pq = pl.pallas_call(
                _attn_body, in_specs=base_in, **attn_specs(c))(g_c, Wf, bfr)
        else:
            h, ps, pq = pl.pallas_call(
                _attn_body_alias, in_specs=base_in + prev_in,
                input_output_aliases={3: 0, 4: 1, 5: 2},
                **attn_specs(c))(g_c, Wf, bfr, h, ps, pq)

    sc_, sh_ = pl.pallas_call(
        _stats_body,
        grid=(1,),
        in_specs=[
            pl.BlockSpec((NBLK, 1, C_), lambda i: (0, 0, 0)),
            pl.BlockSpec((NBLK, 1, C_), lambda i: (0, 0, 0)),
            pl.BlockSpec((1, C_), lambda i: (0, 0)),
            pl.BlockSpec((1, C_), lambda i: (0, 0)),
        ],
        out_specs=[
            pl.BlockSpec((1, C_), lambda i: (0, 0)),
            pl.BlockSpec((1, C_), lambda i: (0, 0)),
        ],
        out_shape=[
            jax.ShapeDtypeStruct((1, C_), jnp.float32),
            jax.ShapeDtypeStruct((1, C_), jnp.float32),
        ],
    )(ps, pq, gamma.reshape(1, C_), beta.reshape(1, C_))

    out = pl.pallas_call(
        _out_body,
        grid=(ROWS // OB,),
        in_specs=[
            pl.BlockSpec((OB, C_), lambda i: (i, 0)),
            pl.BlockSpec((OB, C_), lambda i: (i, 0)),
            pl.BlockSpec((1, C_), lambda i: (0, 0)),
            pl.BlockSpec((1, C_), lambda i: (0, 0)),
        ],
        out_specs=pl.BlockSpec((OB, C_), lambda i: (i, 0)),
        out_shape=jax.ShapeDtypeStruct((ROWS, C_), jnp.float32),
        compiler_params=pltpu.CompilerParams(
            dimension_semantics=("parallel",)),
    )(h, f2, sc_, sh_)

    return out.reshape(B_, N_, C_)


# speedup vs baseline: 1.0155x; 1.0155x over previous
import functools
import math

import jax
import jax.numpy as jnp
from jax.experimental import pallas as pl
from jax.experimental.pallas import tpu as pltpu
from jax.experimental.pallas import tpu_sc as plsc

B_, N_, K_, C_ = 8, 4096, 16, 128
ROWS = B_ * N_
NIDX = ROWS * K_
NEG_INV_S = -1.0 / math.sqrt(C_)

RB = 2048
PB = 512
NBLK = ROWS // PB
OB = 4096
GW = 128


def _proj_body(f_ref, idx_ref, wk_ref, wv_ref, bv_ref, kv_ref, idxo_ref):
    f = f_ref[...]
    kf = jnp.dot(f, wk_ref[...].T, preferred_element_type=jnp.float32)
    vf = jnp.dot(f, wv_ref[...].T, preferred_element_type=jnp.float32)
    kb = jax.lax.bitcast_convert_type(kf * NEG_INV_S, jnp.int32)
    vb = jax.lax.bitcast_convert_type(vf + bv_ref[...], jnp.int32)
    hi = (kb + jnp.int32(0x8000)) & jnp.int32(-65536)
    lo = jax.lax.shift_right_logical(vb + jnp.int32(0x8000), 16)
    kv_ref[...] = hi | lo
    i = pl.program_id(0)
    base = (i * RB) // N_ * N_
    idxo_ref[...] = idx_ref[...] + base


def _attn_body(g_ref, wf_ref, bf_ref, h_ref, ps_ref, pq_ref):
    w = g_ref[0]
    lg = jax.lax.bitcast_convert_type(w & jnp.int32(-65536), jnp.float32)
    v = jax.lax.bitcast_convert_type(
        jax.lax.shift_left(w, jnp.int32(16)), jnp.float32)
    e = jnp.exp(lg)
    s = jnp.sum(e, axis=0)
    t = jnp.max((e - s[None]) * v, axis=0)
    ctx = t / s
    h = jnp.dot(ctx, wf_ref[...].T, preferred_element_type=jnp.float32)
    h = h + bf_ref[...]
    h_ref[...] = h
    ps_ref[...] = jnp.sum(h, axis=0, keepdims=True)[None]
    pq_ref[...] = jnp.sum(h * h, axis=0, keepdims=True)[None]


def _attn_body_alias(g_ref, wf_ref, bf_ref, hp_ref, pp_ref, qp_ref,
                     h_ref, ps_ref, pq_ref):
    del hp_ref, pp_ref, qp_ref
    _attn_body(g_ref, wf_ref, bf_ref, h_ref, ps_ref, pq_ref)


def _stats_body(ps_ref, pq_ref, gm_ref, bt_ref, sc_ref, sh_ref):
    tot = jnp.sum(ps_ref[...], axis=0)
    tot2 = jnp.sum(pq_ref[...], axis=0)
    mean = tot / ROWS
    var = tot2 / ROWS - mean * mean
    sc = gm_ref[...] / jnp.sqrt(var + 1e-5)
    sc_ref[...] = sc
    sh_ref[...] = bt_ref[...] - mean * sc


def _out_body(h_ref, f_ref, sc_ref, sh_ref, o_ref):
    hn = h_ref[...] * sc_ref[...] + sh_ref[...]
    o_ref[...] = f_ref[...] + jnp.where(hn >= 0, hn, 0.2 * hn)


NC = 4
CROWS = ROWS // NC


NW = 32
WIDX = K_ * CROWS // NW
GB = 256
NSTEP = WIDX // GB


def _sc_gather(kv, idxt):
    mesh = plsc.VectorSubcoreMesh(core_axis_name="core",
                                  subcore_axis_name="subcore")

    npb = CROWS // PB
    nbuf = 3

    @functools.partial(
        pl.kernel,
        out_type=jax.ShapeDtypeStruct((npb, K_, PB, C_), jnp.int32),
        mesh=mesh,
        scratch_types=[
            pltpu.VMEM((WIDX,), jnp.int32),
            pltpu.VMEM((nbuf, GB, C_), jnp.int32),
            pltpu.SemaphoreType.DMA,
            pltpu.SemaphoreType.DMA,
            pltpu.SemaphoreType.DMA,
        ])
    def gk(kv_hbm, i_hbm, o_hbm, idx_v, bufs, isem, gsem, osem):
        wid = jax.lax.axis_index("subcore") * 2 + jax.lax.axis_index("core")
        wpk = CROWS // WIDX
        k = wid // wpk
        j0 = (wid % wpk) * WIDX
        pltpu.async_copy(
            i_hbm.at[k, pl.ds(j0, WIDX)], idx_v, isem).wait()

        def mk_g(t):
            return pltpu.make_async_copy(
                kv_hbm.at[idx_v.at[pl.ds(t * GB, GB)]],
                bufs.at[t % nbuf], gsem)

        def mk_o(t):
            pos = j0 + t * GB
            return pltpu.make_async_copy(
                bufs.at[t % nbuf],
                o_hbm.at[pos // PB, k, pl.ds(pos % PB, GB)], osem)

        gs = [None] * NSTEP
        os_ = [None] * NSTEP
        for t in range(NSTEP):
            if t >= nbuf:
                os_[t - nbuf].wait()
            gs[t] = mk_g(t)
            gs[t].start()
            if t >= 1:
                gs[t - 1].wait()
                os_[t - 1] = mk_o(t - 1)
                os_[t - 1].start()
        gs[NSTEP - 1].wait()
        os_[NSTEP - 1] = mk_o(NSTEP - 1)
        os_[NSTEP - 1].start()
        for t in range(max(0, NSTEP - nbuf + 1), NSTEP):
            os_[t].wait()

    return gk(kv, idxt)


def kernel(features, idx, pos, Wq, bq, Wk, bk, Wv, bv, Wf, bf, gamma, beta):
    f2 = features.reshape(ROWS, C_)
    idxt = idx.reshape(ROWS, K_).astype(jnp.int32).T

    cblk = CROWS // RB
    proj_chunk = [
        pl.pallas_call(
            _proj_body,
            grid=(cblk,),
            in_specs=[
                pl.BlockSpec((RB, C_), lambda i, c=c: (i + c * cblk, 0)),
                pl.BlockSpec((K_, RB), lambda i, c=c: (0, i + c * cblk)),
                pl.BlockSpec((C_, C_), lambda i: (0, 0)),
                pl.BlockSpec((C_, C_), lambda i: (0, 0)),
                pl.BlockSpec((1, C_), lambda i: (0, 0)),
            ],
            out_specs=[
                pl.BlockSpec((RB, C_), lambda i: (i, 0)),
                pl.BlockSpec((K_, RB), lambda i: (0, i)),
            ],
            out_shape=[
                jax.ShapeDtypeStruct((CROWS, C_), jnp.int32),
                jax.ShapeDtypeStruct((K_, CROWS), jnp.int32),
            ],
            compiler_params=pltpu.CompilerParams(
                dimension_semantics=("parallel",)),
        )
        for c in range(NC)
    ]

    NBC = NBLK // NC
    attn_out_shape = [
        jax.ShapeDtypeStruct((ROWS, C_), jnp.float32),
        jax.ShapeDtypeStruct((NBLK, 1, C_), jnp.float32),
        jax.ShapeDtypeStruct((NBLK, 1, C_), jnp.float32),
    ]

    def attn_specs(c):
        return dict(
            grid=(NBC,),
            out_specs=[
                pl.BlockSpec((PB, C_), lambda i, c=c: (i + c * NBC, 0)),
                pl.BlockSpec((1, 1, C_), lambda i, c=c: (i + c * NBC, 0, 0)),
                pl.BlockSpec((1, 1, C_), lambda i, c=c: (i + c * NBC, 0, 0)),
            ],
            out_shape=attn_out_shape,
            compiler_params=pltpu.CompilerParams(
                dimension_semantics=("parallel",)),
        )

    base_in = [
        pl.BlockSpec((1, K_, PB, C_), lambda i: (i, 0, 0, 0)),
        pl.BlockSpec((C_, C_), lambda i: (0, 0)),
        pl.BlockSpec((1, C_), lambda i: (0, 0)),
    ]
    prev_in = [
        pl.BlockSpec((8, C_), lambda i: (0, 0)),
        pl.BlockSpec((1, 1, C_), lambda i: (0, 0, 0)),
        pl.BlockSpec((1, 1, C_), lambda i: (0, 0, 0)),
    ]
    h = ps = pq = None
    bfr = bf.reshape(1, C_)
    for c in range(NC):
        kv_c, idxo_c = proj_chunk[c](f2, idxt, Wk, Wv, bv.reshape(1, C_))
        g_c = _sc_gather(kv_c, idxo_c)
        if c == 0:
            h, ps, pq = pl.pallas_call(
                _attn_body, in_specs=base_in, **attn_specs(c))(g_c, Wf, bfr)
        else:
            h, ps, pq = pl.pallas_call(
                _attn_body_alias, in_specs=base_in + prev_in,
                input_output_aliases={3: 0, 4: 1, 5: 2},
                **attn_specs(c))(g_c, Wf, bfr, h, ps, pq)

    sc_, sh_ = pl.pallas_call(
        _stats_body,
        grid=(1,),
        in_specs=[
            pl.BlockSpec((NBLK, 1, C_), lambda i: (0, 0, 0)),
            pl.BlockSpec((NBLK, 1, C_), lambda i: (0, 0, 0)),
            pl.BlockSpec((1, C_), lambda i: (0, 0)),
            pl.BlockSpec((1, C_), lambda i: (0, 0)),
        ],
        out_specs=[
            pl.BlockSpec((1, C_), lambda i: (0, 0)),
            pl.BlockSpec((1, C_), lambda i: (0, 0)),
        ],
        out_shape=[
            jax.ShapeDtypeStruct((1, C_), jnp.float32),
            jax.ShapeDtypeStruct((1, C_), jnp.float32),
        ],
    )(ps, pq, gamma.reshape(1, C_), beta.reshape(1, C_))

    out = pl.pallas_call(
        _out_body,
        grid=(ROWS // OB,),
        in_specs=[
            pl.BlockSpec((OB, C_), lambda i: (i, 0)),
            pl.BlockSpec((OB, C_), lambda i: (i, 0)),
            pl.BlockSpec((1, C_), lambda i: (0, 0)),
            pl.BlockSpec((1, C_), lambda i: (0, 0)),
        ],
        out_specs=pl.BlockSpec((OB, C_), lambda i: (i, 0)),
        out_shape=jax.ShapeDtypeStruct((ROWS, C_), jnp.float32),
        compiler_params=pltpu.CompilerParams(
            dimension_semantics=("parallel",)),
    )(h, f2, sc_, sh_)

    return out.reshape(B_, N_, C_)
